# baseline (speedup 1.0000x reference)
import functools

import jax
import jax.numpy as jnp
from jax import lax
from jax.experimental import pallas as pl
from jax.experimental.pallas import tpu as pltpu
from jax.experimental.pallas import tpu_sc as plsc

N_RIGIDS = 262144
N_RES = 65536
C_FRAME = 128
C_S = 128
N_AA = 21


_BLK_A = 4096


def _proj_body(x_ref, m_ref, g_ref, bln_ref, w_ref, bs_ref, o_ref):
    x = x_ref[...]
    mu = jnp.mean(x, axis=1, keepdims=True)
    xc = x - mu
    var = jnp.mean(xc * xc, axis=1, keepdims=True)
    xn = xc * lax.rsqrt(var + 1e-5)
    xn = xn * g_ref[...] + bln_ref[...]
    y = jnp.dot(xn, w_ref[...], preferred_element_type=jnp.float32)
    o_ref[...] = (y + bs_ref[...]) * m_ref[...]


def _project(x, mask2d, gamma, beta, w, b):
    grid = (N_RIGIDS // _BLK_A,)
    return pl.pallas_call(
        _proj_body,
        grid=grid,
        in_specs=[
            pl.BlockSpec((_BLK_A, C_FRAME), lambda i: (i, 0)),
            pl.BlockSpec((_BLK_A, 1), lambda i: (i, 0)),
            pl.BlockSpec((1, C_FRAME), lambda i: (0, 0)),
            pl.BlockSpec((1, C_FRAME), lambda i: (0, 0)),
            pl.BlockSpec((C_FRAME, C_S), lambda i: (0, 0)),
            pl.BlockSpec((1, C_S), lambda i: (0, 0)),
        ],
        out_specs=pl.BlockSpec((_BLK_A, C_S), lambda i: (i, 0)),
        out_shape=jax.ShapeDtypeStruct((N_RIGIDS, C_S), jnp.float32),
    )(x, mask2d, gamma, beta, w, b)


_NC = 2
_NS = 16
_RANGES = 16
_RNG = N_RES // _RANGES
_RPC = _RANGES // _NC
_TPB = N_RIGIDS // _NS
_STRIPE = _RNG // _NS
_CH = 128
_DUMP = _RNG
_LROWS = (_TPB + _CH) // _CH
_DEPTH = 4

_sc_mesh = plsc.VectorSubcoreMesh(
    core_axis_name="c", subcore_axis_name="s", num_cores=_NC, num_subcores=_NS
)


@functools.partial(
    pl.kernel,
    out_type=jax.ShapeDtypeStruct((N_RES, C_S), jnp.float32),
    mesh=_sc_mesh,
    compiler_params=pltpu.CompilerParams(needs_layout_passes=False),
    scratch_types=[
        pltpu.VMEM_SHARED((_RNG + 8, C_S), jnp.float32),
        pltpu.VMEM((_TPB // 2,), jnp.int32),
        pltpu.VMEM((_LROWS, _CH), jnp.int32),
        pltpu.VMEM((2 * _DEPTH, _CH), jnp.int32),
        pltpu.VMEM((_CH, C_S), jnp.float32),
        pltpu.VMEM((_CH, C_S), jnp.float32),
        pltpu.VMEM((_CH, C_S), jnp.float32),
        pltpu.VMEM((_CH, C_S), jnp.float32),
        pltpu.SemaphoreType.DMA,
    ],
)
def _sc_scatter(val_hbm, idx_hbm, out0_hbm, seq_hbm, acc, idxv, lpack, dec,
                rows0, rows1, rows2, rows3, sem):
    rows = [rows0, rows1, rows2, rows3]
    c = lax.axis_index("c")
    s = lax.axis_index("s")
    tbase = s * _TPB
    lanes = jnp.arange(16, dtype=jnp.int32)
    zero_v = jnp.zeros((16,), jnp.int32)
    one_v = jnp.full((16,), 1, jnp.int32)
    rng_v = jnp.full((16,), _RNG, jnp.int32)
    sh7_v = jnp.full((16,), 7, jnp.int32)
    m127_v = jnp.full((16,), 127, jnp.int32)
    sh13_v = jnp.full((16,), 13, jnp.int32)
    m13_v = jnp.full((16,), (1 << 13) - 1, jnp.int32)
    m18_v = jnp.full((16,), (1 << 18) - 1, jnp.int32)
    dump_v = jnp.full((16,), _DUMP, jnp.int32)
    tbase_v = jnp.full((16,), tbase, jnp.int32)

    def decode(jrow, slot):
        for u in range(_CH // 16):
            p = lpack[jrow, pl.ds(u * 16, 16)]
            dec[2 * slot, pl.ds(u * 16, 16)] = (p >> sh13_v) & m18_v
            dec[2 * slot + 1, pl.ds(u * 16, 16)] = p & m13_v

    def fire(jrow, slot):
        decode(jrow, slot)
        pltpu.async_copy(val_hbm.at[dec.at[2 * slot]], rows[slot], sem)

    def drain_scatter(slot):
        pltpu.make_async_copy(val_hbm.at[dec.at[2 * slot]], rows[slot], sem).wait()
        pltpu.sync_copy(rows[slot], acc.at[dec.at[2 * slot + 1]], add=True)

    def range_body(ri, carry):
        rbase = (ri * _NC + c) * _RNG
        rbase_v = lax.broadcast(rbase, (16,))
        pltpu.sync_copy(
            out0_hbm.at[pl.ds(rbase + s * _STRIPE, _STRIPE)],
            acc.at[pl.ds(s * _STRIPE, _STRIPE)],
        )
        plsc.subcore_barrier()

        cur = jnp.zeros((16,), jnp.int32)
        for h in range(2):
            hbase = tbase + h * (_TPB // 2)
            pltpu.sync_copy(idx_hbm.at[pl.ds(hbase, _TPB // 2)], idxv)
            hbase_v = tbase_v + jnp.full((16,), h * (_TPB // 2), jnp.int32)

            def scan_body(k, cur, hbase_v=hbase_v):
                iv = idxv[pl.ds(k * 16, 16)]
                loc = iv - rbase_v
                m = (loc >= zero_v) & (loc < rng_v)
                pos = cur + plsc.cumsum(m.astype(jnp.int32)) - one_v
                rid = hbase_v + jnp.full((16,), k * 16, jnp.int32) + lanes
                packed = (rid << sh13_v) | loc
                plsc.store_scatter(
                    lpack, [pos >> sh7_v, pos & m127_v], packed, mask=m)
                return cur + plsc.all_reduce_population_count(m)

            cur = lax.fori_loop(0, _TPB // 2 // 16, scan_body, cur)
        ncnt = jnp.max(cur)

        ncnt_v = jnp.full((16,), ncnt, jnp.int32)
        for j2 in range(_CH // 16):
            pp = ncnt_v + (lanes + jnp.full((16,), j2 * 16, jnp.int32))
            plsc.store_scatter(lpack, [pp >> sh7_v, pp & m127_v], dump_v)
        plsc.subcore_barrier()

        nch = (ncnt + (_CH - 1)) >> 7
        for b in range(_DEPTH):
            @pl.when(b < nch)
            def _(b=b):
                fire(b, b)

        def quad_body(q, carry):
            for b in range(_DEPTH):
                j = q * _DEPTH + b

                @pl.when(j < nch)
                def _(j=j, b=b):
                    drain_scatter(b)

                @pl.when(j + _DEPTH < nch)
                def _(j=j, b=b):
                    fire(j + _DEPTH, b)
            return carry

        lax.fori_loop(0, (nch + _DEPTH - 1) >> 2, quad_body, 0)
        plsc.subcore_barrier()

        pltpu.sync_copy(
            acc.at[pl.ds(s * _STRIPE, _STRIPE)],
            seq_hbm.at[pl.ds(rbase + s * _STRIPE, _STRIPE)],
        )
        return carry

    lax.fori_loop(0, _RPC, range_body, 0)


_BLK_C = 8192


def _head_body(x_ref, w_ref, b_ref, o_ref):
    o_ref[...] = (
        jnp.dot(x_ref[...], w_ref[...], preferred_element_type=jnp.float32)
        + b_ref[...]
    )


def _head(seq, w, b):
    grid = (N_RES // _BLK_C,)
    return pl.pallas_call(
        _head_body,
        grid=grid,
        in_specs=[
            pl.BlockSpec((_BLK_C, C_S), lambda i: (i, 0)),
            pl.BlockSpec((C_S, N_AA), lambda i: (0, 0)),
            pl.BlockSpec((1, N_AA), lambda i: (0, 0)),
        ],
        out_specs=pl.BlockSpec((_BLK_C, N_AA), lambda i: (i, 0)),
        out_shape=jax.ShapeDtypeStruct((N_RES, N_AA), jnp.float32),
    )(seq, w, b)


def kernel(rigids_embed_flat, rigids_to_res_idx, rigids_mask, out,
           ln_gamma, ln_beta, W_scatter, b_scatter, W_out, b_out):
    idx = rigids_to_res_idx.astype(jnp.int32)
    val = _project(
        rigids_embed_flat,
        rigids_mask.reshape(N_RIGIDS, 1),
        ln_gamma.reshape(1, C_FRAME),
        ln_beta.reshape(1, C_FRAME),
        W_scatter,
        b_scatter.reshape(1, C_S),
    )
    seq = _sc_scatter(val, idx, out)
    return _head(seq, W_out.astype(jnp.float32), b_out.reshape(1, N_AA))


# speedup vs baseline: 2.0639x; 2.0639x over previous
import functools

import jax
import jax.numpy as jnp
from jax import lax
from jax.experimental import pallas as pl
from jax.experimental.pallas import tpu as pltpu
from jax.experimental.pallas import tpu_sc as plsc

N_RIGIDS = 262144
N_RES = 65536
C_FRAME = 128
C_S = 128
N_AA = 21


_BLK_A = 4096


def _proj_body(x_ref, m_ref, g_ref, bln_ref, w_ref, bs_ref, o_ref):
    x = x_ref[...]
    mu = jnp.mean(x, axis=1, keepdims=True)
    xc = x - mu
    var = jnp.mean(xc * xc, axis=1, keepdims=True)
    xn = xc * lax.rsqrt(var + 1e-5)
    xn = xn * g_ref[...] + bln_ref[...]
    y = jnp.dot(xn, w_ref[...], preferred_element_type=jnp.float32)
    o_ref[...] = (y + bs_ref[...]) * m_ref[...]


def _project(x, mask2d, gamma, beta, w, b):
    grid = (N_RIGIDS // _BLK_A,)
    return pl.pallas_call(
        _proj_body,
        grid=grid,
        in_specs=[
            pl.BlockSpec((_BLK_A, C_FRAME), lambda i: (i, 0)),
            pl.BlockSpec((_BLK_A, 1), lambda i: (i, 0)),
            pl.BlockSpec((1, C_FRAME), lambda i: (0, 0)),
            pl.BlockSpec((1, C_FRAME), lambda i: (0, 0)),
            pl.BlockSpec((C_FRAME, C_S), lambda i: (0, 0)),
            pl.BlockSpec((1, C_S), lambda i: (0, 0)),
        ],
        out_specs=pl.BlockSpec((_BLK_A, C_S), lambda i: (i, 0)),
        out_shape=jax.ShapeDtypeStruct((N_RIGIDS, C_S), jnp.float32),
    )(x, mask2d, gamma, beta, w, b)


_NC = 2
_NS = 16
_RANGES = 8
_RNG = N_RES // _RANGES
_RPC = _RANGES // _NC
_TPB = N_RIGIDS // _NS
_STRIPE = _RNG // _NS
_CH = 128
_DUMP = _RNG
_LROWS = (_TPB + _CH) // _CH + 1

_sc_mesh = plsc.VectorSubcoreMesh(
    core_axis_name="c", subcore_axis_name="s", num_cores=_NC, num_subcores=_NS
)


@functools.partial(
    pl.kernel,
    out_type=jax.ShapeDtypeStruct((N_RES, C_S), jnp.float32),
    mesh=_sc_mesh,
    compiler_params=pltpu.CompilerParams(needs_layout_passes=False),
    scratch_types=[
        pltpu.VMEM_SHARED((_RNG + 8, C_S), jnp.float32),
        pltpu.VMEM((_TPB // 2,), jnp.int32),
        pltpu.VMEM((_LROWS, _CH), jnp.int32),
        pltpu.VMEM((_LROWS, _CH), jnp.int32),
        pltpu.VMEM((_CH, C_S), jnp.float32),
        pltpu.SemaphoreType.DMA,
    ],
)
def _sc_scatter(val_hbm, idx_hbm, out0_hbm, seq_hbm, acc, idxv, lrig, lloc, rows, sem):
    c = lax.axis_index("c")
    s = lax.axis_index("s")
    tbase = s * _TPB
    lanes = jnp.arange(16, dtype=jnp.int32)
    zero_v = jnp.zeros((16,), jnp.int32)
    one_v = jnp.full((16,), 1, jnp.int32)
    rng_v = jnp.full((16,), _RNG, jnp.int32)
    sh7_v = jnp.full((16,), 7, jnp.int32)
    m127_v = jnp.full((16,), 127, jnp.int32)
    dump_v = jnp.full((16,), _DUMP, jnp.int32) + lax.broadcast(s & 7, (16,))
    tbase_v = jnp.full((16,), tbase, jnp.int32)

    for ri in range(_RPC):
        r = ri * _NC + c
        rbase = r * _RNG
        rbase_v = jnp.full((16,), rbase, jnp.int32)
        pltpu.sync_copy(
            out0_hbm.at[pl.ds(rbase + s * _STRIPE, _STRIPE)],
            acc.at[pl.ds(s * _STRIPE, _STRIPE)],
        )
        plsc.subcore_barrier()

        cur = jnp.zeros((16,), jnp.int32)
        for h in range(2):
            hbase = tbase + h * (_TPB // 2)
            pltpu.sync_copy(idx_hbm.at[pl.ds(hbase, _TPB // 2)], idxv)
            hbase_v = tbase_v + jnp.full((16,), h * (_TPB // 2), jnp.int32)

            def scan_body(k, cur, hbase_v=hbase_v):
                iv = idxv[pl.ds(k * 16, 16)]
                loc = iv - rbase_v
                m = (loc >= zero_v) & (loc < rng_v)
                pos = cur + plsc.cumsum(m.astype(jnp.int32)) - one_v
                rid = hbase_v + jnp.full((16,), k * 16, jnp.int32) + lanes
                plsc.store_scatter(lrig, [pos >> sh7_v, pos & m127_v], rid, mask=m)
                plsc.store_scatter(lloc, [pos >> sh7_v, pos & m127_v], loc, mask=m)
                return cur + plsc.all_reduce_population_count(m)

            cur = lax.fori_loop(0, _TPB // 32, scan_body, cur)
        ncnt = jnp.max(cur)

        ncnt_v = jnp.full((16,), ncnt, jnp.int32)
        for j2 in range(_CH // 16):
            pp = ncnt_v + (lanes + jnp.full((16,), j2 * 16, jnp.int32))
            plsc.store_scatter(lloc, [pp >> sh7_v, pp & m127_v], dump_v)
            plsc.store_scatter(lrig, [pp >> sh7_v, pp & m127_v], tbase_v)

        nch = (ncnt + (_CH - 1)) >> 7

        def gs_body(j, carry):
            pltpu.async_copy(val_hbm.at[lrig.at[j]], rows, sem).wait()
            pltpu.sync_copy(rows, acc.at[lloc.at[j]], add=True)
            return carry

        lax.fori_loop(0, nch, gs_body, 0)
        plsc.subcore_barrier()

        pltpu.sync_copy(
            acc.at[pl.ds(s * _STRIPE, _STRIPE)],
            seq_hbm.at[pl.ds(rbase + s * _STRIPE, _STRIPE)],
        )


_BLK_C = 8192


def _head_body(x_ref, w_ref, b_ref, o_ref):
    o_ref[...] = (
        jnp.dot(x_ref[...], w_ref[...], preferred_element_type=jnp.float32)
        + b_ref[...]
    )


def _head(seq, w, b):
    grid = (N_RES // _BLK_C,)
    return pl.pallas_call(
        _head_body,
        grid=grid,
        in_specs=[
            pl.BlockSpec((_BLK_C, C_S), lambda i: (i, 0)),
            pl.BlockSpec((C_S, N_AA), lambda i: (0, 0)),
            pl.BlockSpec((1, N_AA), lambda i: (0, 0)),
        ],
        out_specs=pl.BlockSpec((_BLK_C, N_AA), lambda i: (i, 0)),
        out_shape=jax.ShapeDtypeStruct((N_RES, N_AA), jnp.float32),
    )(seq, w, b)


def kernel(rigids_embed_flat, rigids_to_res_idx, rigids_mask, out,
           ln_gamma, ln_beta, W_scatter, b_scatter, W_out, b_out):
    idx = rigids_to_res_idx.astype(jnp.int32)
    val = _project(
        rigids_embed_flat,
        rigids_mask.reshape(N_RIGIDS, 1),
        ln_gamma.reshape(1, C_FRAME),
        ln_beta.reshape(1, C_FRAME),
        W_scatter,
        b_scatter.reshape(1, C_S),
    )
    seq = _sc_scatter(val, idx, out)
    return _head(seq, W_out.astype(jnp.float32), b_out.reshape(1, N_AA))


# speedup vs baseline: 2.0689x; 1.0024x over previous
import functools

import jax
import jax.numpy as jnp
from jax import lax
from jax.experimental import pallas as pl
from jax.experimental.pallas import tpu as pltpu
from jax.experimental.pallas import tpu_sc as plsc

N_RIGIDS = 262144
N_RES = 65536
C_FRAME = 128
C_S = 128
N_AA = 21


_BLK_A = 4096


def _proj_body(x_ref, m_ref, g_ref, bln_ref, w_ref, bs_ref, o_ref):
    x = x_ref[...]
    mu = jnp.mean(x, axis=1, keepdims=True)
    xc = x - mu
    var = jnp.mean(xc * xc, axis=1, keepdims=True)
    xn = xc * lax.rsqrt(var + 1e-5)
    xn = xn * g_ref[...] + bln_ref[...]
    y = jnp.dot(xn, w_ref[...], preferred_element_type=jnp.float32)
    o_ref[...] = (y + bs_ref[...]) * m_ref[...]


def _project(x, mask2d, gamma, beta, w, b):
    grid = (N_RIGIDS // _BLK_A,)
    return pl.pallas_call(
        _proj_body,
        grid=grid,
        in_specs=[
            pl.BlockSpec((_BLK_A, C_FRAME), lambda i: (i, 0)),
            pl.BlockSpec((_BLK_A, 1), lambda i: (i, 0)),
            pl.BlockSpec((1, C_FRAME), lambda i: (0, 0)),
            pl.BlockSpec((1, C_FRAME), lambda i: (0, 0)),
            pl.BlockSpec((C_FRAME, C_S), lambda i: (0, 0)),
            pl.BlockSpec((1, C_S), lambda i: (0, 0)),
        ],
        out_specs=pl.BlockSpec((_BLK_A, C_S), lambda i: (i, 0)),
        out_shape=jax.ShapeDtypeStruct((N_RIGIDS, C_S), jnp.float32),
    )(x, mask2d, gamma, beta, w, b)


_NC = 2
_NS = 16
_RANGES = 8
_RNG = N_RES // _RANGES
_RPC = _RANGES // _NC
_TPB = N_RIGIDS // _NS
_STRIPE = _RNG // _NS
_CH = 128
_DUMP = _RNG
_LROWS = (_TPB + _CH) // _CH + 1

_sc_mesh = plsc.VectorSubcoreMesh(
    core_axis_name="c", subcore_axis_name="s", num_cores=_NC, num_subcores=_NS
)


@functools.partial(
    pl.kernel,
    out_type=jax.ShapeDtypeStruct((N_RES, C_S), jnp.float32),
    mesh=_sc_mesh,
    compiler_params=pltpu.CompilerParams(needs_layout_passes=False),
    scratch_types=[
        pltpu.VMEM_SHARED((_RNG + 16, C_S), jnp.float32),
        pltpu.VMEM((_TPB // 2,), jnp.int32),
        pltpu.VMEM((_LROWS, _CH), jnp.int32),
        pltpu.VMEM((_LROWS, _CH), jnp.int32),
        pltpu.VMEM((_CH, C_S), jnp.float32),
        pltpu.SemaphoreType.DMA,
    ],
)
def _sc_scatter(val_hbm, idx_hbm, out0_hbm, seq_hbm, acc, idxv, lrig, lloc, rows, sem):
    c = lax.axis_index("c")
    s = lax.axis_index("s")
    tbase = s * _TPB
    lanes = jnp.arange(16, dtype=jnp.int32)
    zero_v = jnp.zeros((16,), jnp.int32)
    one_v = jnp.full((16,), 1, jnp.int32)
    rng_v = jnp.full((16,), _RNG, jnp.int32)
    sh7_v = jnp.full((16,), 7, jnp.int32)
    m127_v = jnp.full((16,), 127, jnp.int32)
    dump_v = jnp.full((16,), _DUMP, jnp.int32) + lax.broadcast(s, (16,))
    tbase_v = jnp.full((16,), tbase, jnp.int32)

    for ri in range(_RPC):
        r = ri * _NC + c
        rbase = r * _RNG
        rbase_v = jnp.full((16,), rbase, jnp.int32)
        pltpu.sync_copy(
            out0_hbm.at[pl.ds(rbase + s * _STRIPE, _STRIPE)],
            acc.at[pl.ds(s * _STRIPE, _STRIPE)],
        )
        plsc.subcore_barrier()

        cur = jnp.zeros((16,), jnp.int32)
        for h in range(2):
            hbase = tbase + h * (_TPB // 2)
            pltpu.sync_copy(idx_hbm.at[pl.ds(hbase, _TPB // 2)], idxv)
            hbase_v = tbase_v + jnp.full((16,), h * (_TPB // 2), jnp.int32)

            def scan_body(k, cur, hbase_v=hbase_v):
                iv = idxv[pl.ds(k * 16, 16)]
                loc = iv - rbase_v
                m = (loc >= zero_v) & (loc < rng_v)
                pos = cur + plsc.cumsum(m.astype(jnp.int32)) - one_v
                rid = hbase_v + jnp.full((16,), k * 16, jnp.int32) + lanes
                plsc.store_scatter(lrig, [pos >> sh7_v, pos & m127_v], rid, mask=m)
                plsc.store_scatter(lloc, [pos >> sh7_v, pos & m127_v], loc, mask=m)
                return cur + plsc.all_reduce_population_count(m)

            cur = lax.fori_loop(0, _TPB // 32, scan_body, cur)
        ncnt = jnp.max(cur)

        ncnt_v = jnp.full((16,), ncnt, jnp.int32)
        for j2 in range(_CH // 16):
            pp = ncnt_v + (lanes + jnp.full((16,), j2 * 16, jnp.int32))
            plsc.store_scatter(lloc, [pp >> sh7_v, pp & m127_v], dump_v)
            plsc.store_scatter(lrig, [pp >> sh7_v, pp & m127_v], tbase_v)

        nch = (ncnt + (_CH - 1)) >> 7

        def gs_body(j, carry):
            pltpu.async_copy(val_hbm.at[lrig.at[j]], rows, sem).wait()
            pltpu.sync_copy(rows, acc.at[lloc.at[j]], add=True)
            return carry

        lax.fori_loop(0, nch, gs_body, 0)
        plsc.subcore_barrier()

        pltpu.sync_copy(
            acc.at[pl.ds(s * _STRIPE, _STRIPE)],
            seq_hbm.at[pl.ds(rbase + s * _STRIPE, _STRIPE)],
        )


_BLK_C = 8192


def _head_body(x_ref, w_ref, b_ref, o_ref):
    o_ref[...] = (
        jnp.dot(x_ref[...], w_ref[...], preferred_element_type=jnp.float32)
        + b_ref[...]
    )


def _head(seq, w, b):
    grid = (N_RES // _BLK_C,)
    return pl.pallas_call(
        _head_body,
        grid=grid,
        in_specs=[
            pl.BlockSpec((_BLK_C, C_S), lambda i: (i, 0)),
            pl.BlockSpec((C_S, N_AA), lambda i: (0, 0)),
            pl.BlockSpec((1, N_AA), lambda i: (0, 0)),
        ],
        out_specs=pl.BlockSpec((_BLK_C, N_AA), lambda i: (i, 0)),
        out_shape=jax.ShapeDtypeStruct((N_RES, N_AA), jnp.float32),
    )(seq, w, b)


def kernel(rigids_embed_flat, rigids_to_res_idx, rigids_mask, out,
           ln_gamma, ln_beta, W_scatter, b_scatter, W_out, b_out):
    idx = rigids_to_res_idx.astype(jnp.int32)
    val = _project(
        rigids_embed_flat,
        rigids_mask.reshape(N_RIGIDS, 1),
        ln_gamma.reshape(1, C_FRAME),
        ln_beta.reshape(1, C_FRAME),
        W_scatter,
        b_scatter.reshape(1, C_S),
    )
    seq = _sc_scatter(val, idx, out)
    return _head(seq, W_out.astype(jnp.float32), b_out.reshape(1, N_AA))
